# baseline (speedup 1.0000x reference)
import functools
import math

import jax
import jax.numpy as jnp
from jax import lax
from jax.experimental import pallas as pl
from jax.experimental.pallas import tpu as pltpu
from jax.experimental.pallas import tpu_sc as plsc

NC = 2
NS = 16
NW = NC * NS
L = 16


def _emb_body(scale, n_chunks, chunk, d_model,
              t_hbm, p_hbm, s_hbm, tok_hbm, pos_hbm, seg_hbm, out_hbm,
              tv, pv, sv, segb,
              a0, b0, o0, a1, b1, o1,
              gs0, gs1, os0, os1):
    per_w = n_chunks * chunk
    cid = lax.axis_index("c")
    sid = lax.axis_index("s")
    wid = sid * NC + cid
    base = wid * per_w

    sets = ((a0, b0, o0, gs0, os0), (a1, b1, o1, gs1, os1))

    pltpu.sync_copy(t_hbm.at[pl.ds(base, per_w)], tv)
    pltpu.sync_copy(p_hbm.at[pl.ds(base, per_w)], pv)
    pltpu.sync_copy(s_hbm.at[pl.ds(base, per_w)], sv)
    pltpu.sync_copy(seg_hbm, segb)
    for j in range(d_model // L):
        sl = pl.ds(j * L, L)
        segb[1, sl] = segb[1, sl] - segb[0, sl]

    def issue_gathers(g, bufs):
        a, b, _, gsem, _ = bufs
        off = g * chunk
        pltpu.async_copy(tok_hbm.at[tv.at[pl.ds(off, chunk)]], a, gsem)
        pltpu.async_copy(pos_hbm.at[pv.at[pl.ds(off, chunk)]], b, gsem)

    issue_gathers(0, sets[0])
    issue_gathers(1, sets[1])

    @pl.loop(0, n_chunks, step=2)
    def _pipeline(i):
        for k in range(2):
            a, b, o, gsem, osem = sets[k]
            g = i + k
            off = g * chunk
            pltpu.make_async_copy(
                tok_hbm.at[tv.at[pl.ds(off, chunk)]], a, gsem).wait()
            pltpu.make_async_copy(
                pos_hbm.at[pv.at[pl.ds(off, chunk)]], b, gsem).wait()

            @pl.when(g >= 2)
            def _():
                pltpu.make_async_copy(
                    o, out_hbm.at[pl.ds(base + off, chunk)], osem).wait()

            svec = sv[pl.ds(off, chunk)]

            def combine_row(r, c2):
                s_r = jnp.take(svec, lax.broadcast(r, (L,)), mode="wrap")
                s_f = s_r.astype(jnp.float32)
                for j in range(d_model // L):
                    sl = pl.ds(j * L, L)
                    segv = segb[0, sl] + s_f * segb[1, sl]
                    o[r, sl] = (a[r, sl] + b[r, sl] + segv) * scale
                return c2

            lax.fori_loop(0, chunk, combine_row, 0)

            pltpu.async_copy(o, out_hbm.at[pl.ds(base + off, chunk)], osem)

            @pl.when(g + 2 < n_chunks)
            def _():
                issue_gathers(g + 2, sets[k])

    for k in range(2):
        o, osem = sets[k][2], sets[k][4]
        pltpu.make_async_copy(o, out_hbm.at[pl.ds(base, chunk)], osem).wait()


@jax.jit
def kernel(t, p, s, tok_w, pos_w, seg_w):
    b, s_len = t.shape
    d_model = tok_w.shape[1]
    n_seg = seg_w.shape[0]
    assert n_seg == 2
    total = b * s_len
    scale = math.sqrt(float(d_model))

    chunk = L
    assert total % (NW * chunk) == 0
    n_chunks = total // (NW * chunk)
    assert n_chunks % 2 == 0 and n_chunks >= 4

    tf = t.reshape(total).astype(jnp.int32)
    pf = p.reshape(total).astype(jnp.int32)
    sf = s.reshape(total).astype(jnp.int32)

    mesh = plsc.VectorSubcoreMesh(core_axis_name="c", subcore_axis_name="s",
                                  num_cores=NC, num_subcores=NS)
    body = functools.partial(_emb_body, scale, n_chunks, chunk, d_model)
    per_w = n_chunks * chunk
    buf = pltpu.VMEM((chunk, d_model), jnp.float32)
    run = pl.kernel(
        body,
        out_type=jax.ShapeDtypeStruct((total, d_model), jnp.float32),
        mesh=mesh,
        scratch_types=[
            pltpu.VMEM((per_w,), jnp.int32),
            pltpu.VMEM((per_w,), jnp.int32),
            pltpu.VMEM((per_w,), jnp.int32),
            pltpu.VMEM((n_seg, d_model), jnp.float32),
            buf, buf, buf, buf, buf, buf,
            pltpu.SemaphoreType.DMA,
            pltpu.SemaphoreType.DMA,
            pltpu.SemaphoreType.DMA,
            pltpu.SemaphoreType.DMA,
        ],
    )
    out = run(tf, pf, sf, tok_w, pos_w, seg_w)
    return out.reshape(b, s_len, d_model)


# speedup vs baseline: 1.5296x; 1.5296x over previous
import functools
import math

import jax
import jax.numpy as jnp
from jax import lax
from jax.experimental import pallas as pl
from jax.experimental.pallas import tpu as pltpu
from jax.experimental.pallas import tpu_sc as plsc

NC = 2
NS = 16
NW = NC * NS
L = 16
SEG_REP = 1024


def _emb_body(scale, n_chunks, chunk, d_model,
              t_hbm, p_hbm, s_hbm, tok_hbm, pos_hbm, seg_hbm, out_hbm,
              tv, pv, sv,
              a0, b0, c0, o0, a1, b1, c1, o1,
              gs0, gs1, os0, os1):
    per_w = n_chunks * chunk
    cid = lax.axis_index("c")
    sid = lax.axis_index("s")
    wid = sid * NC + cid
    base = wid * per_w

    sets = ((a0, b0, c0, o0, gs0, os0), (a1, b1, c1, o1, gs1, os1))

    pltpu.sync_copy(t_hbm.at[pl.ds(base, per_w)], tv)
    pltpu.sync_copy(p_hbm.at[pl.ds(base, per_w)], pv)
    pltpu.sync_copy(s_hbm.at[pl.ds(base, per_w)], sv)

    def issue_gathers(g, bufs):
        a, b, c, _, gsem, _ = bufs
        off = g * chunk
        pltpu.async_copy(tok_hbm.at[tv.at[pl.ds(off, chunk)]], a, gsem)
        pltpu.async_copy(pos_hbm.at[pv.at[pl.ds(off, chunk)]], b, gsem)
        pltpu.async_copy(seg_hbm.at[sv.at[pl.ds(off, chunk)]], c, gsem)

    issue_gathers(0, sets[0])
    issue_gathers(1, sets[1])

    @pl.loop(0, n_chunks, step=2)
    def _pipeline(i):
        for k in range(2):
            a, b, c, o, gsem, osem = sets[k]
            g = i + k
            off = g * chunk
            pltpu.make_async_copy(
                tok_hbm.at[tv.at[pl.ds(off, chunk)]], a, gsem).wait()
            pltpu.make_async_copy(
                pos_hbm.at[pv.at[pl.ds(off, chunk)]], b, gsem).wait()
            pltpu.make_async_copy(
                seg_hbm.at[sv.at[pl.ds(off, chunk)]], c, gsem).wait()

            @pl.when(g >= 2)
            def _():
                pltpu.make_async_copy(
                    o, out_hbm.at[pl.ds(base + off, chunk)], osem).wait()

            def combine_row(r, c2):
                for j in range(d_model // L):
                    sl = pl.ds(j * L, L)
                    o[r, sl] = (a[r, sl] + b[r, sl] + c[r, sl]) * scale
                return c2

            lax.fori_loop(0, chunk, combine_row, 0)

            pltpu.async_copy(o, out_hbm.at[pl.ds(base + off, chunk)], osem)

            @pl.when(g + 2 < n_chunks)
            def _():
                issue_gathers(g + 2, sets[k])

    for k in range(2):
        o, osem = sets[k][3], sets[k][5]
        pltpu.make_async_copy(o, out_hbm.at[pl.ds(base, chunk)], osem).wait()


@jax.jit
def kernel(t, p, s, tok_w, pos_w, seg_w):
    b, s_len = t.shape
    d_model = tok_w.shape[1]
    n_seg = seg_w.shape[0]
    total = b * s_len
    scale = math.sqrt(float(d_model))

    chunk = 8
    assert total % (NW * chunk) == 0
    n_chunks = total // (NW * chunk)
    assert n_chunks % 2 == 0 and n_chunks >= 4

    tf = t.reshape(total).astype(jnp.int32)
    pf = p.reshape(total).astype(jnp.int32)
    seg_rep = jnp.tile(seg_w, (SEG_REP, 1))
    sf = (s.reshape(total).astype(jnp.int32)
          + n_seg * (jnp.arange(total, dtype=jnp.int32) % SEG_REP))

    mesh = plsc.VectorSubcoreMesh(core_axis_name="c", subcore_axis_name="s",
                                  num_cores=NC, num_subcores=NS)
    body = functools.partial(_emb_body, scale, n_chunks, chunk, d_model)
    per_w = n_chunks * chunk
    buf = pltpu.VMEM((chunk, d_model), jnp.float32)
    run = pl.kernel(
        body,
        out_type=jax.ShapeDtypeStruct((total, d_model), jnp.float32),
        mesh=mesh,
        scratch_types=[
            pltpu.VMEM((per_w,), jnp.int32),
            pltpu.VMEM((per_w,), jnp.int32),
            pltpu.VMEM((per_w,), jnp.int32),
            buf, buf, buf, buf, buf, buf, buf, buf,
            pltpu.SemaphoreType.DMA,
            pltpu.SemaphoreType.DMA,
            pltpu.SemaphoreType.DMA,
            pltpu.SemaphoreType.DMA,
        ],
    )
    out = run(tf, pf, sf, tok_w, pos_w, seg_rep)
    return out.reshape(b, s_len, d_model)


# speedup vs baseline: 1.7502x; 1.1442x over previous
import functools
import math

import jax
import jax.numpy as jnp
from jax import lax
from jax.experimental import pallas as pl
from jax.experimental.pallas import tpu as pltpu
from jax.experimental.pallas import tpu_sc as plsc

NC = 2
NS = 16
NW = NC * NS
L = 16


def _emb_body(scale, n_chunks, chunk, d_model,
              t_hbm, c_hbm, tok_hbm, comb_hbm, out_hbm,
              tv, cv,
              a0, b0, o0, a1, b1, o1,
              gs0, gs1, os0, os1):
    per_w = n_chunks * chunk
    cid = lax.axis_index("c")
    sid = lax.axis_index("s")
    wid = sid * NC + cid
    base = wid * per_w

    sets = ((a0, b0, o0, gs0, os0), (a1, b1, o1, gs1, os1))

    pltpu.sync_copy(t_hbm.at[pl.ds(base, per_w)], tv)
    pltpu.sync_copy(c_hbm.at[pl.ds(base, per_w)], cv)

    def issue_gathers(g, bufs):
        a, b, _, gsem, _ = bufs
        off = g * chunk
        pltpu.async_copy(tok_hbm.at[tv.at[pl.ds(off, chunk)]], a, gsem)
        pltpu.async_copy(comb_hbm.at[cv.at[pl.ds(off, chunk)]], b, gsem)

    issue_gathers(0, sets[0])
    issue_gathers(1, sets[1])

    @pl.loop(0, n_chunks, step=2)
    def _pipeline(i):
        for k in range(2):
            a, b, o, gsem, osem = sets[k]
            g = i + k
            off = g * chunk
            pltpu.make_async_copy(
                tok_hbm.at[tv.at[pl.ds(off, chunk)]], a, gsem).wait()
            pltpu.make_async_copy(
                comb_hbm.at[cv.at[pl.ds(off, chunk)]], b, gsem).wait()

            @pl.when(g >= 2)
            def _():
                pltpu.make_async_copy(
                    o, out_hbm.at[pl.ds(base + off, chunk)], osem).wait()

            def combine_row(r, c2):
                for j in range(d_model // L):
                    sl = pl.ds(j * L, L)
                    o[r, sl] = (a[r, sl] + b[r, sl]) * scale
                return c2

            lax.fori_loop(0, chunk, combine_row, 0)

            pltpu.async_copy(o, out_hbm.at[pl.ds(base + off, chunk)], osem)

            @pl.when(g + 2 < n_chunks)
            def _():
                issue_gathers(g + 2, sets[k])

    for k in range(2):
        o, osem = sets[k][2], sets[k][4]
        pltpu.make_async_copy(o, out_hbm.at[pl.ds(base, chunk)], osem).wait()


@jax.jit
def kernel(t, p, s, tok_w, pos_w, seg_w):
    b, s_len = t.shape
    d_model = tok_w.shape[1]
    n_seg = seg_w.shape[0]
    total = b * s_len
    scale = math.sqrt(float(d_model))

    chunk = 16
    assert total % (NW * chunk) == 0
    n_chunks = total // (NW * chunk)
    assert n_chunks % 2 == 0 and n_chunks >= 4

    tf = t.reshape(total).astype(jnp.int32)
    comb = (pos_w[:, None, :] + seg_w[None, :, :]).reshape(-1, d_model)
    cf = (p.reshape(total).astype(jnp.int32) * n_seg
          + s.reshape(total).astype(jnp.int32))

    mesh = plsc.VectorSubcoreMesh(core_axis_name="c", subcore_axis_name="s",
                                  num_cores=NC, num_subcores=NS)
    body = functools.partial(_emb_body, scale, n_chunks, chunk, d_model)
    per_w = n_chunks * chunk
    buf = pltpu.VMEM((chunk, d_model), jnp.float32)
    run = pl.kernel(
        body,
        out_type=jax.ShapeDtypeStruct((total, d_model), jnp.float32),
        mesh=mesh,
        scratch_types=[
            pltpu.VMEM((per_w,), jnp.int32),
            pltpu.VMEM((per_w,), jnp.int32),
            buf, buf, buf, buf, buf, buf,
            pltpu.SemaphoreType.DMA,
            pltpu.SemaphoreType.DMA,
            pltpu.SemaphoreType.DMA,
            pltpu.SemaphoreType.DMA,
        ],
    )
    out = run(tf, cf, tok_w, comb)
    return out.reshape(b, s_len, d_model)
